# baseline (speedup 1.0000x reference)
import functools

import jax
import jax.numpy as jnp
from jax import lax
from jax.experimental import pallas as pl
from jax.experimental.pallas import tpu as pltpu
from jax.experimental.pallas import tpu_sc as plsc

D_DIM = 64
BATCH = 4096
HIST = 50

NC = 2
NS = 16
NW = NC * NS
BB = BATCH // NW
L = 16

_mesh = plsc.VectorSubcoreMesh(core_axis_name="c", subcore_axis_name="s")


@functools.partial(
    pl.kernel,
    out_type=jax.ShapeDtypeStruct((HIST, 8, NW, 1024), jnp.float32),
    mesh=_mesh,
    scratch_types=[
        pltpu.VMEM((HIST, BB), jnp.int32),
        pltpu.VMEM((BB, D_DIM), jnp.float32),
        pltpu.VMEM((BB, D_DIM), jnp.float32),
        pltpu.VMEM((8, 1024), jnp.float32),
        pltpu.VMEM((8, 1024), jnp.float32),
        pltpu.SemaphoreType.DMA,
        pltpu.SemaphoreType.DMA,
        pltpu.SemaphoreType.DMA,
        pltpu.SemaphoreType.DMA,
    ],
    compiler_params=pltpu.CompilerParams(
        use_tc_tiling_on_sc=False, needs_layout_passes=False),
)
def _gather_kernel(xt_hbm, table_hbm, out_hbm, idx_v, rb0, rb1, sb0, sb1,
                   gs0, gs1, ws0, ws1):
    wid = lax.axis_index("s") * NC + lax.axis_index("c")
    pltpu.sync_copy(xt_hbm.at[:, pl.ds(wid * BB, BB)], idx_v)
    iota = lax.iota(jnp.int32, L)

    def fire_g(h, rb, sem):
        pltpu.async_copy(table_hbm.at[idx_v.at[h]], rb, sem)

    def drain_g(h, rb, sem):
        pltpu.make_async_copy(table_hbm.at[idx_v.at[h]], rb, sem).wait()

    def fire_w(h, sb, sem):
        pltpu.async_copy(sb, out_hbm.at[h, :, wid], sem)

    def drain_w(h, sb, sem):
        pltpu.make_async_copy(sb, out_hbm.at[h, :, wid], sem).wait()

    def transpose(rb, sb):
        @pl.loop(0, 8)
        def _tr(tr):
            for s in range(8):
                d = tr * 8 + s
                dvec = jnp.full((L,), 0, jnp.int32) + d
                for g in range(BB // L):
                    rvec = iota + g * L
                    v = plsc.load_gather(rb, [rvec, dvec])
                    sb[tr, pl.ds(s * 128 + g * L, L)] = v

    fire_g(0, rb0, gs0)

    @pl.loop(0, HIST // 2)
    def _pair(t):
        h0 = 2 * t
        h1 = h0 + 1
        fire_g(h1, rb1, gs1)
        drain_g(h0, rb0, gs0)

        @pl.when(t > 0)
        def _():
            drain_w(h0 - 2, sb0, ws0)

        transpose(rb0, sb0)
        fire_w(h0, sb0, ws0)

        @pl.when(t < HIST // 2 - 1)
        def _():
            fire_g(h1 + 1, rb0, gs0)

        drain_g(h1, rb1, gs1)

        @pl.when(t > 0)
        def _():
            drain_w(h1 - 2, sb1, ws1)

        transpose(rb1, sb1)
        fire_w(h1, sb1, ws1)

    drain_w(HIST - 2, sb0, ws0)
    drain_w(HIST - 1, sb1, ws1)


def kernel(x, item_emb_mat):
    xt = x.T.astype(jnp.int32)
    out_t = _gather_kernel(xt, item_emb_mat)
    out5 = out_t.reshape(HIST, 8, NW, 8, 128)
    return out5.transpose(2, 4, 0, 1, 3).reshape(BATCH, HIST, D_DIM)


# speedup vs baseline: 1.0654x; 1.0654x over previous
import functools

import jax
import jax.numpy as jnp
from jax import lax
from jax.experimental import pallas as pl
from jax.experimental.pallas import tpu as pltpu
from jax.experimental.pallas import tpu_sc as plsc

D_DIM = 64
BATCH = 4096
HIST = 50

NC = 2
NS = 16
NW = NC * NS
BB = BATCH // NW
L = 16

_mesh = plsc.VectorSubcoreMesh(core_axis_name="c", subcore_axis_name="s")


@functools.partial(
    pl.kernel,
    out_type=jax.ShapeDtypeStruct((HIST, 8, NW, 1024), jnp.float32),
    mesh=_mesh,
    scratch_types=[
        pltpu.VMEM((HIST, BB), jnp.int32),
        pltpu.VMEM((BB, D_DIM), jnp.float32),
        pltpu.VMEM((BB, D_DIM), jnp.float32),
        pltpu.VMEM((8 * 1024,), jnp.float32),
        pltpu.VMEM((8 * 1024,), jnp.float32),
        pltpu.SemaphoreType.DMA,
        pltpu.SemaphoreType.DMA,
        pltpu.SemaphoreType.DMA,
        pltpu.SemaphoreType.DMA,
    ],
    compiler_params=pltpu.CompilerParams(
        use_tc_tiling_on_sc=False, needs_layout_passes=False),
)
def _gather_kernel(xt_hbm, table_hbm, out_hbm, idx_v, rb0, rb1, sb0, sb1,
                   gs0, gs1, ws0, ws1):
    wid = lax.axis_index("s") * NC + lax.axis_index("c")
    pltpu.sync_copy(xt_hbm.at[:, pl.ds(wid * BB, BB)], idx_v)
    iota = lax.iota(jnp.int32, L)
    d16 = [iota + L * k for k in range(D_DIM // L)]
    gb = [(d % 8) * 128 + (d // 8) * 1024 for d in d16]

    def fire_g(h, rb, sem):
        pltpu.async_copy(table_hbm.at[idx_v.at[h]], rb, sem)

    def drain_g(h, rb, sem):
        pltpu.make_async_copy(table_hbm.at[idx_v.at[h]], rb, sem).wait()

    def fire_w(h, sb, sem):
        for tr in range(8):
            pltpu.async_copy(
                sb.at[pl.ds(tr * 1024, 1024)], out_hbm.at[h, tr, wid], sem)

    def drain_w(h, sb, sem):
        for tr in range(8):
            pltpu.make_async_copy(
                sb.at[pl.ds(tr * 1024, 1024)], out_hbm.at[h, tr, wid],
                sem).wait()

    def transpose(rb, sb):
        for b in range(BB):
            for k in range(D_DIM // L):
                v = rb[b, pl.ds(k * L, L)]
                plsc.store_scatter(sb, [gb[k] + b], v)

    fire_g(0, rb0, gs0)

    @pl.loop(0, HIST // 2)
    def _pair(t):
        h0 = 2 * t
        h1 = h0 + 1
        fire_g(h1, rb1, gs1)
        drain_g(h0, rb0, gs0)

        @pl.when(t > 0)
        def _():
            drain_w(h0 - 2, sb0, ws0)

        transpose(rb0, sb0)
        fire_w(h0, sb0, ws0)

        @pl.when(t < HIST // 2 - 1)
        def _():
            fire_g(h1 + 1, rb0, gs0)

        drain_g(h1, rb1, gs1)

        @pl.when(t > 0)
        def _():
            drain_w(h1 - 2, sb1, ws1)

        transpose(rb1, sb1)
        fire_w(h1, sb1, ws1)

    drain_w(HIST - 2, sb0, ws0)
    drain_w(HIST - 1, sb1, ws1)


def kernel(x, item_emb_mat):
    xt = x.T.astype(jnp.int32)
    out_t = _gather_kernel(xt, item_emb_mat)
    out5 = out_t.reshape(HIST, 8, NW, 8, 128)
    return out5.transpose(2, 4, 0, 1, 3).reshape(BATCH, HIST, D_DIM)


# speedup vs baseline: 1.0709x; 1.0051x over previous
import functools

import jax
import jax.numpy as jnp
from jax import lax
from jax.experimental import pallas as pl
from jax.experimental.pallas import tpu as pltpu
from jax.experimental.pallas import tpu_sc as plsc

D_DIM = 64
BATCH = 4096
HIST = 50

NC = 2
NS = 16
NW = NC * NS
BB = BATCH // NW
L = 16

_mesh = plsc.VectorSubcoreMesh(core_axis_name="c", subcore_axis_name="s")


@functools.partial(
    pl.kernel,
    out_type=jax.ShapeDtypeStruct((HIST, 8, NW, 1024), jnp.float32),
    mesh=_mesh,
    scratch_types=[
        pltpu.VMEM((HIST, BB), jnp.int32),
        pltpu.VMEM((BB, D_DIM), jnp.float32),
        pltpu.VMEM((BB, D_DIM), jnp.float32),
        pltpu.VMEM((8 * 1024,), jnp.float32),
        pltpu.VMEM((8 * 1024,), jnp.float32),
        pltpu.SemaphoreType.DMA,
        pltpu.SemaphoreType.DMA,
        pltpu.SemaphoreType.DMA,
        pltpu.SemaphoreType.DMA,
    ],
    compiler_params=pltpu.CompilerParams(
        use_tc_tiling_on_sc=False, needs_layout_passes=False),
)
def _gather_kernel(xt_hbm, table_hbm, out_hbm, idx_v, rb0, rb1, sb0, sb1,
                   gs0, gs1, ws0, ws1):
    wid = lax.axis_index("s") * NC + lax.axis_index("c")
    pltpu.sync_copy(xt_hbm.at[:, pl.ds(wid * BB, BB)], idx_v)
    iota = lax.iota(jnp.int32, L)
    d16 = [iota + L * k for k in range(D_DIM // L)]
    gb = [(d % 8) * 128 + (d // 8) * 1024 for d in d16]

    def fire_g(h, rb, sem):
        pltpu.async_copy(table_hbm.at[idx_v.at[h]], rb, sem)

    def drain_g(h, rb, sem):
        pltpu.make_async_copy(table_hbm.at[idx_v.at[h]], rb, sem).wait()

    def fire_w(h, sb, sem):
        for tr in range(8):
            pltpu.async_copy(
                sb.at[pl.ds(tr * 1024, 1024)], out_hbm.at[h, tr, wid], sem)

    def drain_w(h, sb, sem):
        for tr in range(8):
            pltpu.make_async_copy(
                sb.at[pl.ds(tr * 1024, 1024)], out_hbm.at[h, tr, wid],
                sem).wait()

    def transpose(rb, sb):
        @pl.loop(0, BB, unroll=8)
        def _row(b):
            for k in range(D_DIM // L):
                v = rb[b, pl.ds(k * L, L)]
                plsc.store_scatter(sb, [gb[k] + b], v)

    fire_g(0, rb0, gs0)

    @pl.loop(0, HIST // 2)
    def _pair(t):
        h0 = 2 * t
        h1 = h0 + 1
        fire_g(h1, rb1, gs1)
        drain_g(h0, rb0, gs0)

        @pl.when(t > 0)
        def _():
            drain_w(h0 - 2, sb0, ws0)

        transpose(rb0, sb0)
        fire_w(h0, sb0, ws0)

        @pl.when(t < HIST // 2 - 1)
        def _():
            fire_g(h1 + 1, rb0, gs0)

        drain_g(h1, rb1, gs1)

        @pl.when(t > 0)
        def _():
            drain_w(h1 - 2, sb1, ws1)

        transpose(rb1, sb1)
        fire_w(h1, sb1, ws1)

    drain_w(HIST - 2, sb0, ws0)
    drain_w(HIST - 1, sb1, ws1)


def kernel(x, item_emb_mat):
    xt = x.T.astype(jnp.int32)
    out_t = _gather_kernel(xt, item_emb_mat)
    out5 = out_t.reshape(HIST, 8, NW, 8, 128)
    return out5.transpose(2, 4, 0, 1, 3).reshape(BATCH, HIST, D_DIM)


# speedup vs baseline: 1.0728x; 1.0018x over previous
import functools

import jax
import jax.numpy as jnp
from jax import lax
from jax.experimental import pallas as pl
from jax.experimental.pallas import tpu as pltpu
from jax.experimental.pallas import tpu_sc as plsc

D_DIM = 64
BATCH = 4096
HIST = 50

NC = 2
NS = 16
NW = NC * NS
BB = BATCH // NW
L = 16
NB = 5
NT = HIST // NB

_mesh = plsc.VectorSubcoreMesh(core_axis_name="c", subcore_axis_name="s")


@functools.partial(
    pl.kernel,
    out_type=jax.ShapeDtypeStruct((HIST, 8, NW, 1024), jnp.float32),
    mesh=_mesh,
    scratch_types=(
        [pltpu.VMEM((HIST, BB), jnp.int32)]
        + [pltpu.VMEM((BB, D_DIM), jnp.float32) for _ in range(NB)]
        + [pltpu.VMEM((8, 1024), jnp.float32) for _ in range(NB)]
        + [pltpu.SemaphoreType.DMA for _ in range(2 * NB)]
    ),
    compiler_params=pltpu.CompilerParams(
        use_tc_tiling_on_sc=False, needs_layout_passes=False),
)
def _gather_kernel(xt_hbm, table_hbm, out_hbm, idx_v, *bufs):
    rb = bufs[:NB]
    sb = bufs[NB:2 * NB]
    gs = bufs[2 * NB:3 * NB]
    ws = bufs[3 * NB:4 * NB]
    wid = lax.axis_index("s") * NC + lax.axis_index("c")
    pltpu.sync_copy(xt_hbm.at[:, pl.ds(wid * BB, BB)], idx_v)
    iota = lax.iota(jnp.int32, L)
    d16 = [iota + L * k for k in range(D_DIM // L)]
    rvec = [d // 8 for d in d16]
    cvec = [(d % 8) * 128 for d in d16]

    def fire_g(h, j):
        pltpu.async_copy(table_hbm.at[idx_v.at[h]], rb[j], gs[j])

    def drain_g(h, j):
        pltpu.make_async_copy(table_hbm.at[idx_v.at[h]], rb[j], gs[j]).wait()

    def fire_w(h, j):
        pltpu.async_copy(sb[j], out_hbm.at[h, :, wid], ws[j])

    def drain_w(h, j):
        pltpu.make_async_copy(sb[j], out_hbm.at[h, :, wid], ws[j]).wait()

    def transpose(j):
        @pl.loop(0, BB, unroll=8)
        def _row(b):
            for k in range(D_DIM // L):
                v = rb[j][b, pl.ds(k * L, L)]
                plsc.store_scatter(sb[j], [rvec[k], cvec[k] + b], v)

    for j in range(NB):
        fire_g(j, j)

    @pl.loop(0, NT)
    def _outer(t):
        for j in range(NB):
            h = NB * t + j
            drain_g(h, j)

            @pl.when(t > 0)
            def _():
                drain_w(h - NB, j)

            transpose(j)
            fire_w(h, j)

            @pl.when(t < NT - 1)
            def _():
                fire_g(h + NB, j)

    for j in range(NB):
        drain_w(HIST - NB + j, j)


def kernel(x, item_emb_mat):
    xt = x.T.astype(jnp.int32)
    out_t = _gather_kernel(xt, item_emb_mat)
    out5 = out_t.reshape(HIST, 8, NW, 8, 128)
    return out5.transpose(2, 4, 0, 1, 3).reshape(BATCH, HIST, D_DIM)


# speedup vs baseline: 1.4251x; 1.3284x over previous
import functools

import jax
import jax.numpy as jnp
from jax import lax
from jax.experimental import pallas as pl
from jax.experimental.pallas import tpu as pltpu
from jax.experimental.pallas import tpu_sc as plsc

D_DIM = 64
BATCH = 4096
HIST = 50

NC = 2
NS = 16
NW = NC * NS
BB = BATCH // NW
L = 16
NB = 5
NT = HIST // NB

_mesh = plsc.VectorSubcoreMesh(core_axis_name="c", subcore_axis_name="s")


@functools.partial(
    pl.kernel,
    out_type=jax.ShapeDtypeStruct((HIST, 8, NW, 1024), jnp.float32),
    mesh=_mesh,
    scratch_types=(
        [pltpu.VMEM((HIST, BB), jnp.int32)]
        + [pltpu.VMEM((BB, D_DIM), jnp.float32) for _ in range(NB)]
        + [pltpu.VMEM((8, 1024), jnp.float32) for _ in range(NB)]
        + [pltpu.SemaphoreType.DMA for _ in range(2 * NB)]
    ),
    compiler_params=pltpu.CompilerParams(
        use_tc_tiling_on_sc=False, needs_layout_passes=False),
)
def _gather_kernel(xt_hbm, table_hbm, out_hbm, idx_v, *bufs):
    rb = bufs[:NB]
    sb = bufs[NB:2 * NB]
    gs = bufs[2 * NB:3 * NB]
    ws = bufs[3 * NB:4 * NB]
    wid = lax.axis_index("s") * NC + lax.axis_index("c")
    pltpu.sync_copy(xt_hbm.at[:, pl.ds(wid * BB, BB)], idx_v)
    iota = lax.iota(jnp.int32, L)
    drow = [2 * k + iota // 8 for k in range(D_DIM // L)]
    dcol = [(iota % 8) * 128 + iota for _ in range(D_DIM // L)]

    def fire_g(h, j):
        pltpu.async_copy(table_hbm.at[idx_v.at[h]], rb[j], gs[j])

    def drain_g(h, j):
        pltpu.make_async_copy(table_hbm.at[idx_v.at[h]], rb[j], gs[j]).wait()

    def fire_w(h, j):
        pltpu.async_copy(sb[j], out_hbm.at[h, :, wid], ws[j])

    def drain_w(h, j):
        pltpu.make_async_copy(sb[j], out_hbm.at[h, :, wid], ws[j]).wait()

    dlo = [iota + L * k for k in range(D_DIM // L)]

    def transpose(j):
        @pl.loop(0, BB // L, unroll=4)
        def _blk(g):
            b0 = g * L
            bvec = iota + b0
            for k in range(D_DIM // L):
                v = plsc.load_gather(rb[j], [bvec, dlo[k]])
                plsc.store_scatter(sb[j], [drow[k], dcol[k] + b0], v)

    for j in range(NB):
        fire_g(j, j)

    @pl.loop(0, NT)
    def _outer(t):
        for j in range(NB):
            h = NB * t + j
            drain_g(h, j)

            @pl.when(t > 0)
            def _():
                drain_w(h - NB, j)

            transpose(j)
            fire_w(h, j)

            @pl.when(t < NT - 1)
            def _():
                fire_g(h + NB, j)

    for j in range(NB):
        drain_w(HIST - NB + j, j)


def kernel(x, item_emb_mat):
    xt = x.T.astype(jnp.int32)
    out_t = _gather_kernel(xt, item_emb_mat)
    out5 = out_t.reshape(HIST, 8, NW, 8, 128)
    return out5.transpose(2, 4, 0, 1, 3).reshape(BATCH, HIST, D_DIM)
